# baseline (speedup 1.0000x reference)
import functools

import jax
import jax.numpy as jnp
from jax import lax
from jax.experimental import pallas as pl
from jax.experimental.pallas import tpu as pltpu
from jax.experimental.pallas import tpu_sc as plsc

_NUM_WORKERS = 32


@functools.lru_cache(maxsize=None)
def _sc_upsample(n, h, w, c, dtype_name):
    dtype = jnp.dtype(dtype_name)
    h_per = h // _NUM_WORKERS
    mesh = plsc.VectorSubcoreMesh(core_axis_name="c", subcore_axis_name="s")

    @functools.partial(
        pl.kernel,
        mesh=mesh,
        out_type=jax.ShapeDtypeStruct((n, 2 * h, 2 * w, c), dtype),
        scratch_types=[
            pltpu.VMEM((w, c), dtype),
            pltpu.VMEM((2 * w, c), dtype),
        ],
    )
    def k(img_hbm, out_hbm, x_v, y_v):
        cid = lax.axis_index("c")
        sid = lax.axis_index("s")
        wid = sid * 2 + cid
        h0 = wid * h_per

        for b in range(n):

            def row_body(t, carry):
                hh = h0 + t
                pltpu.sync_copy(img_hbm.at[b, hh], x_v)

                def w_body(u, inner):
                    for j in range(c // 16):
                        v = x_v[u, pl.ds(16 * j, 16)]
                        y_v[2 * u, pl.ds(16 * j, 16)] = v
                        y_v[2 * u + 1, pl.ds(16 * j, 16)] = v
                    return inner

                lax.fori_loop(0, w, w_body, 0, unroll=False)
                pltpu.sync_copy(y_v, out_hbm.at[b, 2 * hh])
                pltpu.sync_copy(y_v, out_hbm.at[b, 2 * hh + 1])
                return carry

            lax.fori_loop(0, h_per, row_body, 0, unroll=False)

    return k


def kernel(img):
    n, h, w, c = img.shape
    return _sc_upsample(n, h, w, c, img.dtype.name)(img)


# speedup vs baseline: 1.0873x; 1.0873x over previous
import functools

import jax
import jax.numpy as jnp
from jax import lax
from jax.experimental import pallas as pl
from jax.experimental.pallas import tpu as pltpu
from jax.experimental.pallas import tpu_sc as plsc

_NUM_WORKERS = 32


@functools.lru_cache(maxsize=None)
def _sc_upsample(n, h, w, c, dtype_name):
    dtype = jnp.dtype(dtype_name)
    h_per = h // _NUM_WORKERS
    rows = n * h_per
    mesh = plsc.VectorSubcoreMesh(core_axis_name="c", subcore_axis_name="s")

    @functools.partial(
        pl.kernel,
        mesh=mesh,
        out_type=jax.ShapeDtypeStruct((n, 2 * h, 2 * w, c), dtype),
        scratch_types=[
            pltpu.VMEM((2 * w, c), dtype),
            pltpu.VMEM((2 * w, c), dtype),
            pltpu.SemaphoreType.DMA,
            pltpu.SemaphoreType.DMA,
            pltpu.SemaphoreType.DMA,
            pltpu.SemaphoreType.DMA,
        ],
    )
    def k(img_hbm, out_hbm, y0, y1, in0, in1, out0, out1):
        cid = lax.axis_index("c")
        sid = lax.axis_index("s")
        wid = sid * 2 + cid
        h0 = wid * h_per

        bufs = (y0, y1)
        in_sems = (in0, in1)
        out_sems = (out0, out1)

        def src_row(t):
            b, r = divmod(t, h_per)
            return b, h0 + r

        def start_in(t):
            b, hh = src_row(t)
            p = t % 2
            return pltpu.async_copy(
                img_hbm.at[b, hh], bufs[p].at[pl.ds(0, w)], in_sems[p])

        def expand(buf):
            def w_body(u, carry):
                ud = (w - 1) - u
                for j in range(c // 16):
                    v = buf[ud, pl.ds(16 * j, 16)]
                    buf[2 * ud, pl.ds(16 * j, 16)] = v
                    buf[2 * ud + 1, pl.ds(16 * j, 16)] = v
                return carry

            lax.fori_loop(0, w, w_body, 0, unroll=False)

        pending_out = [None, None]
        pending_in = [None, None]

        pending_in[0] = start_in(0)
        for t in range(rows):
            p = t % 2
            q = 1 - p
            if pending_out[q] is not None:
                for cp in pending_out[q]:
                    cp.wait()
                pending_out[q] = None
            if t + 1 < rows:
                pending_in[q] = start_in(t + 1)
            pending_in[p].wait()
            expand(bufs[p])
            b, hh = src_row(t)
            pending_out[p] = (
                pltpu.async_copy(bufs[p], out_hbm.at[b, 2 * hh], out_sems[p]),
                pltpu.async_copy(bufs[p], out_hbm.at[b, 2 * hh + 1], out_sems[p]),
            )
        for po in pending_out:
            if po is not None:
                for cp in po:
                    cp.wait()

    return k


def kernel(img):
    n, h, w, c = img.shape
    return _sc_upsample(n, h, w, c, img.dtype.name)(img)


# speedup vs baseline: 1.0874x; 1.0000x over previous
import functools

import jax
import jax.numpy as jnp
from jax import lax
from jax.experimental import pallas as pl
from jax.experimental.pallas import tpu as pltpu
from jax.experimental.pallas import tpu_sc as plsc

_NUM_WORKERS = 32


@functools.lru_cache(maxsize=None)
def _sc_upsample(n, h, w, c, dtype_name):
    dtype = jnp.dtype(dtype_name)
    h_per = h // _NUM_WORKERS
    rows = n * h_per
    mesh = plsc.VectorSubcoreMesh(core_axis_name="c", subcore_axis_name="s")

    @functools.partial(
        pl.kernel,
        mesh=mesh,
        compiler_params=pltpu.CompilerParams(use_tc_tiling_on_sc=True),
        out_type=jax.ShapeDtypeStruct((n, 2 * h, 2 * w, c), dtype),
        scratch_types=[
            pltpu.VMEM((2 * w, c), dtype),
            pltpu.VMEM((2 * w, c), dtype),
            pltpu.SemaphoreType.DMA,
            pltpu.SemaphoreType.DMA,
            pltpu.SemaphoreType.DMA,
            pltpu.SemaphoreType.DMA,
        ],
    )
    def k(img_hbm, out_hbm, y0, y1, in0, in1, out0, out1):
        cid = lax.axis_index("c")
        sid = lax.axis_index("s")
        wid = sid * 2 + cid
        h0 = wid * h_per

        bufs = (y0, y1)
        in_sems = (in0, in1)
        out_sems = (out0, out1)

        def src_row(t):
            b, r = divmod(t, h_per)
            return b, h0 + r

        def start_in(t):
            b, hh = src_row(t)
            p = t % 2
            return pltpu.async_copy(
                img_hbm.at[b, hh], bufs[p].at[pl.ds(0, w)], in_sems[p])

        def expand(buf):
            def w_body(u, carry):
                ud = (w - 1) - u
                for j in range(c // 16):
                    v = buf[ud, pl.ds(16 * j, 16)]
                    buf[2 * ud, pl.ds(16 * j, 16)] = v
                    buf[2 * ud + 1, pl.ds(16 * j, 16)] = v
                return carry

            lax.fori_loop(0, w, w_body, 0, unroll=False)

        pending_out = [None, None]
        pending_in = [None, None]

        pending_in[0] = start_in(0)
        for t in range(rows):
            p = t % 2
            q = 1 - p
            if pending_out[q] is not None:
                for cp in pending_out[q]:
                    cp.wait()
                pending_out[q] = None
            if t + 1 < rows:
                pending_in[q] = start_in(t + 1)
            pending_in[p].wait()
            expand(bufs[p])
            b, hh = src_row(t)
            pending_out[p] = (
                pltpu.async_copy(bufs[p], out_hbm.at[b, 2 * hh], out_sems[p]),
                pltpu.async_copy(bufs[p], out_hbm.at[b, 2 * hh + 1], out_sems[p]),
            )
        for po in pending_out:
            if po is not None:
                for cp in po:
                    cp.wait()

    return k


def kernel(img):
    n, h, w, c = img.shape
    return _sc_upsample(n, h, w, c, img.dtype.name)(img)


# speedup vs baseline: 1.3349x; 1.2277x over previous
import jax
import jax.numpy as jnp
from jax.experimental import pallas as pl

_ROWS_PER_BLOCK = 32


def _upsample_block(x_ref, o_ref):
    x = x_ref[0]
    ib, w, c = x.shape
    y = jnp.broadcast_to(x[:, None, :, None, :], (ib, 2, w, 2, c))
    o_ref[0] = y.reshape(2 * ib, 2 * w, c)


def kernel(img):
    n, h, w, c = img.shape
    ib = _ROWS_PER_BLOCK
    return pl.pallas_call(
        _upsample_block,
        grid=(n, h // ib),
        in_specs=[pl.BlockSpec((1, ib, w, c), lambda b, i: (b, i, 0, 0))],
        out_specs=pl.BlockSpec((1, 2 * ib, 2 * w, c), lambda b, i: (b, i, 0, 0)),
        out_shape=jax.ShapeDtypeStruct((n, 2 * h, 2 * w, c), img.dtype),
    )(img)
